# baseline (speedup 1.0000x reference)
import functools

import jax
import jax.numpy as jnp
from jax import lax
from jax.experimental import pallas as pl
from jax.experimental.pallas import tpu as pltpu
from jax.experimental.pallas import tpu_sc as plsc

N = 10000
D = 128
G = 64
DO = 512
NC, NS, L = 2, 16, 16
NW = NC * NS
K = 128
NP = 10240
ROWS_PT = NP // NS
DEGW = 128
BM = 512


def _sc_mesh():
    return plsc.VectorSubcoreMesh(
        core_axis_name="c", subcore_axis_name="s", num_cores=NC, num_subcores=NS)


def _num_chunks(num_edges):
    return -(-num_edges // (NW * K))


NPR = NP // 128
DROWS_PT = 8


def _deg_body(cpt, dst_hbm, eye_hbm, degp_hbm, didx_v, cidx_v, rows_v, zrow_v,
              deg_sp, sem):
    c = lax.axis_index("c")
    s = lax.axis_index("s")
    wid = c * NS + s
    base = wid * (cpt * K)

    def fill_zero(r, _):
        for j in range(DEGW // L):
            zrow_v[r, pl.ds(j * L, L)] = jnp.zeros((L,), jnp.float32)
        return 0
    lax.fori_loop(0, DROWS_PT, fill_zero, 0)

    def fill_rows_zero(r, _):
        for j in range(DEGW // L):
            rows_v[r, pl.ds(j * L, L)] = jnp.zeros((L,), jnp.float32)
        return 0
    lax.fori_loop(0, K, fill_rows_zero, 0)

    @pl.when(s < NPR // DROWS_PT)
    def _():
        pltpu.sync_copy(zrow_v, deg_sp.at[pl.ds(s * DROWS_PT, DROWS_PT)])
    plsc.subcore_barrier()

    def chunk(i, _):
        pltpu.sync_copy(dst_hbm.at[pl.ds(base + i * K, K)], didx_v)
        for j in range(K // L):
            v = didx_v[pl.ds(j * L, L)]
            cidx_v[pl.ds(j * L, L)] = lax.bitwise_and(v, 127)
            didx_v[pl.ds(j * L, L)] = lax.shift_right_logical(v, 7)
        pltpu.sync_copy(eye_hbm.at[cidx_v], rows_v)
        pltpu.sync_copy(rows_v, deg_sp.at[didx_v], add=True)
        return 0
    lax.fori_loop(0, cpt, chunk, 0)
    plsc.subcore_barrier()

    @pl.when(s < NPR // DROWS_PT)
    def _():
        pltpu.sync_copy(deg_sp.at[pl.ds(s * DROWS_PT, DROWS_PT)], zrow_v)
        pltpu.sync_copy(zrow_v, degp_hbm.at[c, pl.ds(s * DROWS_PT, DROWS_PT)])


def _make_deg_call(cpt):
    return functools.partial(
        pl.kernel,
        out_type=jax.ShapeDtypeStruct((NC, NPR, DEGW), jnp.float32),
        mesh=_sc_mesh(),
        scratch_types=[
            pltpu.VMEM((K,), jnp.int32),
            pltpu.VMEM((K,), jnp.int32),
            pltpu.VMEM((K, DEGW), jnp.float32),
            pltpu.VMEM((DROWS_PT, DEGW), jnp.float32),
            pltpu.VMEM_SHARED((NPR, DEGW), jnp.float32),
            pltpu.SemaphoreType.DMA,
        ],
    )(functools.partial(_deg_body, cpt))


def _msg_body(cpt, src_hbm, dst_hbm, y_hbm, accp_hbm,
              sidx_v, didx_v, rows_v, acc_sp, sem):
    c = lax.axis_index("c")
    s = lax.axis_index("s")
    wid = c * NS + s
    base = wid * (cpt * K)

    def zrow(r, _):
        for j in range(D // L):
            rows_v[r, pl.ds(j * L, L)] = jnp.zeros((L,), jnp.float32)
        return 0
    lax.fori_loop(0, K, zrow, 0)
    for j in range(ROWS_PT // K):
        pltpu.sync_copy(rows_v, acc_sp.at[pl.ds(s * ROWS_PT + j * K, K)])
    plsc.subcore_barrier()

    def chunk(i, _):
        pltpu.sync_copy(src_hbm.at[pl.ds(base + i * K, K)], sidx_v)
        pltpu.sync_copy(dst_hbm.at[pl.ds(base + i * K, K)], didx_v)
        pltpu.sync_copy(y_hbm.at[sidx_v], rows_v)
        pltpu.sync_copy(rows_v, acc_sp.at[didx_v], add=True)
        return 0
    lax.fori_loop(0, cpt, chunk, 0)
    plsc.subcore_barrier()
    for j in range(ROWS_PT // K):
        r0 = s * ROWS_PT + j * K
        pltpu.sync_copy(acc_sp.at[pl.ds(r0, K)], rows_v)
        pltpu.sync_copy(rows_v, accp_hbm.at[c, pl.ds(r0, K)])


def _make_msg_call(cpt):
    return functools.partial(
        pl.kernel,
        out_type=jax.ShapeDtypeStruct((NC, NP, D), jnp.float32),
        mesh=_sc_mesh(),
        scratch_types=[
            pltpu.VMEM((K,), jnp.int32),
            pltpu.VMEM((K,), jnp.int32),
            pltpu.VMEM((K, D), jnp.float32),
            pltpu.VMEM_SHARED((NP, D), jnp.float32),
            pltpu.SemaphoreType.DMA,
        ],
    )(functools.partial(_msg_body, cpt))


def _dinv_of(degp_ref):
    d = degp_ref[0] + degp_ref[1] + 1.0
    return lax.rsqrt(d)


def _scale_body(x_ref, w_ref, degp_ref, y_ref):
    dinv = _dinv_of(degp_ref)
    y_ref[...] = dinv * jnp.dot(x_ref[...], w_ref[...],
                                preferred_element_type=jnp.float32)


def _layer2_body(accp_ref, y1_ref, degp_ref, b1_ref, w2_ref, y2_ref):
    dinv = _dinv_of(degp_ref)
    h = jnp.maximum(
        dinv * (accp_ref[0] + accp_ref[1] + y1_ref[...]) + b1_ref[...], 0.0)
    y2_ref[...] = dinv * jnp.dot(h, w2_ref[...],
                                 preferred_element_type=jnp.float32)


def _final_body(nb, accp_ref, y2_ref, degp_ref, b2_ref, batch_ref, wo_ref,
                bo_ref, out_ref, sums, cnts):
    m = pl.program_id(0)

    @pl.when(m == 0)
    def _():
        sums[...] = jnp.zeros_like(sums)
        cnts[...] = jnp.zeros_like(cnts)

    dinv = _dinv_of(degp_ref)
    h = jnp.maximum(
        dinv * (accp_ref[0] + accp_ref[1] + y2_ref[...]) + b2_ref[...], 0.0)
    oh = (batch_ref[...] == lax.broadcasted_iota(jnp.int32, (G, BM), 0)
          ).astype(jnp.float32)
    sums[...] += jnp.dot(oh, h, preferred_element_type=jnp.float32)
    cnts[...] = cnts[...] + jnp.sum(oh, axis=1, keepdims=True)

    @pl.when(m == nb - 1)
    def _():
        pooled = sums[...] / jnp.maximum(cnts[...], 1.0)
        out_ref[...] = jnp.dot(pooled, wo_ref[...],
                               preferred_element_type=jnp.float32) + bo_ref[...]


_NB = NP // BM

_scale_call = pl.pallas_call(
    _scale_body,
    grid=(_NB,),
    in_specs=[
        pl.BlockSpec((BM, D), lambda i: (i, 0)),
        pl.BlockSpec((D, D), lambda i: (0, 0)),
        pl.BlockSpec((NC, BM, 1), lambda i: (0, i, 0)),
    ],
    out_specs=pl.BlockSpec((BM, D), lambda i: (i, 0)),
    out_shape=jax.ShapeDtypeStruct((NP, D), jnp.float32),
)

_layer2_call = pl.pallas_call(
    _layer2_body,
    grid=(_NB,),
    in_specs=[
        pl.BlockSpec((NC, BM, D), lambda i: (0, i, 0)),
        pl.BlockSpec((BM, D), lambda i: (i, 0)),
        pl.BlockSpec((NC, BM, 1), lambda i: (0, i, 0)),
        pl.BlockSpec((1, D), lambda i: (0, 0)),
        pl.BlockSpec((D, D), lambda i: (0, 0)),
    ],
    out_specs=pl.BlockSpec((BM, D), lambda i: (i, 0)),
    out_shape=jax.ShapeDtypeStruct((NP, D), jnp.float32),
)

_final_call = pl.pallas_call(
    functools.partial(_final_body, _NB),
    grid=(_NB,),
    in_specs=[
        pl.BlockSpec((NC, BM, D), lambda i: (0, i, 0)),
        pl.BlockSpec((BM, D), lambda i: (i, 0)),
        pl.BlockSpec((NC, BM, 1), lambda i: (0, i, 0)),
        pl.BlockSpec((1, D), lambda i: (0, 0)),
        pl.BlockSpec((1, BM), lambda i: (0, i)),
        pl.BlockSpec((D, DO), lambda i: (0, 0)),
        pl.BlockSpec((1, DO), lambda i: (0, 0)),
    ],
    out_specs=pl.BlockSpec((G, DO), lambda i: (0, 0)),
    out_shape=jax.ShapeDtypeStruct((G, DO), jnp.float32),
    scratch_shapes=[
        pltpu.VMEM((G, D), jnp.float32),
        pltpu.VMEM((G, D), jnp.float32),
    ],
)


def kernel(x, edge_index, batch, W1, b1, W2, b2, Wo, bo):
    num_edges = edge_index.shape[1]
    cpt = _num_chunks(num_edges)
    ep = NW * cpt * K
    src = edge_index[0].astype(jnp.int32)
    dst = edge_index[1].astype(jnp.int32)
    pad = jnp.full((ep - num_edges,), N, jnp.int32)
    src_p = jnp.concatenate([src, pad])
    dst_p = jnp.concatenate([dst, pad])
    n = x.shape[0]
    x_p = jnp.concatenate([x, jnp.zeros((NP - n, D), jnp.float32)])
    batch_p = jnp.concatenate(
        [batch.astype(jnp.int32), jnp.full((NP - n,), G, jnp.int32)])[None, :]

    deg_call = _make_deg_call(cpt)
    msg_call = _make_msg_call(cpt)

    eye = jnp.eye(DEGW, dtype=jnp.float32)
    degp = deg_call(dst_p, eye).reshape(NC, NP, 1)
    y1 = _scale_call(x_p, W1, degp)
    acc1 = msg_call(src_p, dst_p, y1)
    y2 = _layer2_call(acc1, y1, degp, b1[None, :], W2)
    acc2 = msg_call(src_p, dst_p, y2)
    out = _final_call(acc2, y2, degp, b2[None, :], batch_p, Wo, bo[None, :])
    return out


# speedup vs baseline: 1.0470x; 1.0470x over previous
import functools

import jax
import jax.numpy as jnp
from jax import lax
from jax.experimental import pallas as pl
from jax.experimental.pallas import tpu as pltpu
from jax.experimental.pallas import tpu_sc as plsc

N = 10000
D = 128
G = 64
DO = 512
NC, NS, L = 2, 16, 16
NW = NC * NS
K = 128
NP = 10240
ROWS_PT = NP // NS
DEGW = 128
BM = 512


def _sc_mesh():
    return plsc.VectorSubcoreMesh(
        core_axis_name="c", subcore_axis_name="s", num_cores=NC, num_subcores=NS)


def _num_chunks(num_edges):
    return -(-num_edges // (NW * K))


NPR = NP // 128
DROWS_PT = 8


def _deg_body(cpt, dst_hbm, eye_hbm, degp_hbm, didx_v, cidx_v, rows_v, zrow_v,
              deg_sp, sem):
    c = lax.axis_index("c")
    s = lax.axis_index("s")
    wid = c * NS + s
    base = wid * (cpt * K)

    def fill_zero(r, _):
        for j in range(DEGW // L):
            zrow_v[r, pl.ds(j * L, L)] = jnp.zeros((L,), jnp.float32)
        return 0
    lax.fori_loop(0, DROWS_PT, fill_zero, 0)

    def fill_rows_zero(r, _):
        for j in range(DEGW // L):
            rows_v[r, pl.ds(j * L, L)] = jnp.zeros((L,), jnp.float32)
        return 0
    lax.fori_loop(0, K, fill_rows_zero, 0)

    @pl.when(s < NPR // DROWS_PT)
    def _():
        pltpu.sync_copy(zrow_v, deg_sp.at[pl.ds(s * DROWS_PT, DROWS_PT)])
    plsc.subcore_barrier()

    def chunk(i, _):
        pltpu.sync_copy(dst_hbm.at[pl.ds(base + i * K, K)], didx_v)
        for j in range(K // L):
            v = didx_v[pl.ds(j * L, L)]
            cidx_v[pl.ds(j * L, L)] = lax.bitwise_and(v, 127)
            didx_v[pl.ds(j * L, L)] = lax.shift_right_logical(v, 7)
        pltpu.sync_copy(eye_hbm.at[cidx_v], rows_v)
        pltpu.sync_copy(rows_v, deg_sp.at[didx_v], add=True)
        return 0
    lax.fori_loop(0, cpt, chunk, 0)
    plsc.subcore_barrier()

    @pl.when(s < NPR // DROWS_PT)
    def _():
        pltpu.sync_copy(deg_sp.at[pl.ds(s * DROWS_PT, DROWS_PT)], zrow_v)
        pltpu.sync_copy(zrow_v, degp_hbm.at[c, pl.ds(s * DROWS_PT, DROWS_PT)])


def _make_deg_call(cpt):
    return functools.partial(
        pl.kernel,
        out_type=jax.ShapeDtypeStruct((NC, NPR, DEGW), jnp.float32),
        mesh=_sc_mesh(),
        scratch_types=[
            pltpu.VMEM((K,), jnp.int32),
            pltpu.VMEM((K,), jnp.int32),
            pltpu.VMEM((K, DEGW), jnp.float32),
            pltpu.VMEM((DROWS_PT, DEGW), jnp.float32),
            pltpu.VMEM_SHARED((NPR, DEGW), jnp.float32),
            pltpu.SemaphoreType.DMA,
        ],
    )(functools.partial(_deg_body, cpt))


KM = 48
MBUF = 4


def _num_chunks_msg(num_edges):
    cpt = -(-num_edges // (NW * KM))
    return -(-cpt // MBUF) * MBUF


def _msg_body(cpt, src_hbm, dst_hbm, y_hbm, accp_hbm,
              s0, s1, s2, s3, d0, d1, d2, d3, r0, r1, r2, r3, acc_sp,
              i0, i1, i2, i3, g0, g1, g2, g3, c0, c1, c2, c3):
    sidx = (s0, s1, s2, s3)
    didx = (d0, d1, d2, d3)
    rows = (r0, r1, r2, r3)
    isem = (i0, i1, i2, i3)
    gsem = (g0, g1, g2, g3)
    ssem = (c0, c1, c2, c3)
    c = lax.axis_index("c")
    s = lax.axis_index("s")
    wid = c * NS + s
    base = wid * (cpt * KM)

    def zrow(r, _):
        for j in range(D // L):
            r0[r, pl.ds(j * L, L)] = jnp.zeros((L,), jnp.float32)
        return 0
    lax.fori_loop(0, KM, zrow, 0)
    for j in range(ROWS_PT // KM):
        pltpu.sync_copy(r0, acc_sp.at[pl.ds(s * ROWS_PT + j * KM, KM)])
    rem = ROWS_PT % KM
    if rem:
        pltpu.sync_copy(r0.at[pl.ds(0, rem)],
                        acc_sp.at[pl.ds(s * ROWS_PT + (ROWS_PT // KM) * KM, rem)])
    plsc.subcore_barrier()

    def i_start(t, b):
        pltpu.async_copy(src_hbm.at[pl.ds(base + t * KM, KM)], sidx[b], isem[b])
        pltpu.async_copy(dst_hbm.at[pl.ds(base + t * KM, KM)], didx[b], isem[b])

    def i_wait(t, b):
        pltpu.make_async_copy(src_hbm.at[pl.ds(base + t * KM, KM)], sidx[b],
                              isem[b]).wait()
        pltpu.make_async_copy(dst_hbm.at[pl.ds(base + t * KM, KM)], didx[b],
                              isem[b]).wait()

    def g_start(t, b):
        pltpu.async_copy(y_hbm.at[sidx[b]], rows[b], gsem[b])

    def g_wait(t, b):
        pltpu.make_async_copy(y_hbm.at[sidx[b]], rows[b], gsem[b]).wait()

    def s_start(t, b):
        pltpu.async_copy(rows[b], acc_sp.at[didx[b]], ssem[b], add=True)

    def s_wait(t, b):
        pltpu.make_async_copy(rows[b], acc_sp.at[didx[b]], ssem[b]).wait()

    i_start(0, 0)
    i_start(1, 1)
    i_wait(0, 0)
    g_start(0, 0)

    def grp(g, _):
        for b in range(MBUF):
            t = g * MBUF + b
            u = t + 2
            bu = (b + 2) % MBUF
            v = t + 1
            bv = (b + 1) % MBUF

            @pl.when(jnp.logical_and(u < cpt, u >= MBUF))
            def _():
                s_wait(u - MBUF, bu)
                i_start(u, bu)

            @pl.when(jnp.logical_and(u < cpt, u < MBUF))
            def _():
                i_start(u, bu)

            @pl.when(v < cpt)
            def _():
                i_wait(v, bv)
                g_start(v, bv)

            g_wait(t, b)
            s_start(t, b)
        return 0
    lax.fori_loop(0, cpt // MBUF, grp, 0)
    for b in range(MBUF):
        s_wait(cpt - MBUF + b, b)
    plsc.subcore_barrier()
    for j in range(ROWS_PT // KM):
        rr = s * ROWS_PT + j * KM
        pltpu.sync_copy(acc_sp.at[pl.ds(rr, KM)], r0)
        pltpu.sync_copy(r0, accp_hbm.at[c, pl.ds(rr, KM)])
    if ROWS_PT % KM:
        rr = s * ROWS_PT + (ROWS_PT // KM) * KM
        rem = ROWS_PT % KM
        pltpu.sync_copy(acc_sp.at[pl.ds(rr, rem)], r0.at[pl.ds(0, rem)])
        pltpu.sync_copy(r0.at[pl.ds(0, rem)], accp_hbm.at[c, pl.ds(rr, rem)])


def _make_msg_call(cpt):
    return functools.partial(
        pl.kernel,
        out_type=jax.ShapeDtypeStruct((NC, NP, D), jnp.float32),
        mesh=_sc_mesh(),
        scratch_types=(
            [pltpu.VMEM((KM,), jnp.int32)] * (2 * MBUF)
            + [pltpu.VMEM((KM, D), jnp.float32)] * MBUF
            + [pltpu.VMEM_SHARED((NP, D), jnp.float32)]
            + [pltpu.SemaphoreType.DMA] * (3 * MBUF)
        ),
    )(functools.partial(_msg_body, cpt))


def _dinv_of(degp_ref):
    d = degp_ref[0] + degp_ref[1] + 1.0
    return lax.rsqrt(d)


def _scale_body(x_ref, w_ref, degp_ref, y_ref):
    dinv = _dinv_of(degp_ref)
    y_ref[...] = dinv * jnp.dot(x_ref[...], w_ref[...],
                                preferred_element_type=jnp.float32)


def _layer2_body(accp_ref, y1_ref, degp_ref, b1_ref, w2_ref, y2_ref):
    dinv = _dinv_of(degp_ref)
    h = jnp.maximum(
        dinv * (accp_ref[0] + accp_ref[1] + y1_ref[...]) + b1_ref[...], 0.0)
    y2_ref[...] = dinv * jnp.dot(h, w2_ref[...],
                                 preferred_element_type=jnp.float32)


def _final_body(nb, accp_ref, y2_ref, degp_ref, b2_ref, batch_ref, wo_ref,
                bo_ref, out_ref, sums, cnts):
    m = pl.program_id(0)

    @pl.when(m == 0)
    def _():
        sums[...] = jnp.zeros_like(sums)
        cnts[...] = jnp.zeros_like(cnts)

    dinv = _dinv_of(degp_ref)
    h = jnp.maximum(
        dinv * (accp_ref[0] + accp_ref[1] + y2_ref[...]) + b2_ref[...], 0.0)
    oh = (batch_ref[...] == lax.broadcasted_iota(jnp.int32, (G, BM), 0)
          ).astype(jnp.float32)
    sums[...] += jnp.dot(oh, h, preferred_element_type=jnp.float32)
    cnts[...] = cnts[...] + jnp.sum(oh, axis=1, keepdims=True)

    @pl.when(m == nb - 1)
    def _():
        pooled = sums[...] / jnp.maximum(cnts[...], 1.0)
        out_ref[...] = jnp.dot(pooled, wo_ref[...],
                               preferred_element_type=jnp.float32) + bo_ref[...]


_NB = NP // BM

_scale_call = pl.pallas_call(
    _scale_body,
    grid=(_NB,),
    in_specs=[
        pl.BlockSpec((BM, D), lambda i: (i, 0)),
        pl.BlockSpec((D, D), lambda i: (0, 0)),
        pl.BlockSpec((NC, BM, 1), lambda i: (0, i, 0)),
    ],
    out_specs=pl.BlockSpec((BM, D), lambda i: (i, 0)),
    out_shape=jax.ShapeDtypeStruct((NP, D), jnp.float32),
)

_layer2_call = pl.pallas_call(
    _layer2_body,
    grid=(_NB,),
    in_specs=[
        pl.BlockSpec((NC, BM, D), lambda i: (0, i, 0)),
        pl.BlockSpec((BM, D), lambda i: (i, 0)),
        pl.BlockSpec((NC, BM, 1), lambda i: (0, i, 0)),
        pl.BlockSpec((1, D), lambda i: (0, 0)),
        pl.BlockSpec((D, D), lambda i: (0, 0)),
    ],
    out_specs=pl.BlockSpec((BM, D), lambda i: (i, 0)),
    out_shape=jax.ShapeDtypeStruct((NP, D), jnp.float32),
)

_final_call = pl.pallas_call(
    functools.partial(_final_body, _NB),
    grid=(_NB,),
    in_specs=[
        pl.BlockSpec((NC, BM, D), lambda i: (0, i, 0)),
        pl.BlockSpec((BM, D), lambda i: (i, 0)),
        pl.BlockSpec((NC, BM, 1), lambda i: (0, i, 0)),
        pl.BlockSpec((1, D), lambda i: (0, 0)),
        pl.BlockSpec((1, BM), lambda i: (0, i)),
        pl.BlockSpec((D, DO), lambda i: (0, 0)),
        pl.BlockSpec((1, DO), lambda i: (0, 0)),
    ],
    out_specs=pl.BlockSpec((G, DO), lambda i: (0, 0)),
    out_shape=jax.ShapeDtypeStruct((G, DO), jnp.float32),
    scratch_shapes=[
        pltpu.VMEM((G, D), jnp.float32),
        pltpu.VMEM((G, D), jnp.float32),
    ],
)


def kernel(x, edge_index, batch, W1, b1, W2, b2, Wo, bo):
    num_edges = edge_index.shape[1]
    cpt = _num_chunks(num_edges)
    cptm = _num_chunks_msg(num_edges)
    src = edge_index[0].astype(jnp.int32)
    dst = edge_index[1].astype(jnp.int32)
    pad = jnp.full((NW * cpt * K - num_edges,), N, jnp.int32)
    padm = jnp.full((NW * cptm * KM - num_edges,), N, jnp.int32)
    dst_p = jnp.concatenate([dst, pad])
    src_m = jnp.concatenate([src, padm])
    dst_m = jnp.concatenate([dst, padm])
    n = x.shape[0]
    x_p = jnp.concatenate([x, jnp.zeros((NP - n, D), jnp.float32)])
    batch_p = jnp.concatenate(
        [batch.astype(jnp.int32), jnp.full((NP - n,), G, jnp.int32)])[None, :]

    deg_call = _make_deg_call(cpt)
    msg_call = _make_msg_call(cptm)

    eye = jnp.eye(DEGW, dtype=jnp.float32)
    degp = deg_call(dst_p, eye).reshape(NC, NP, 1)
    y1 = _scale_call(x_p, W1, degp)
    acc1 = msg_call(src_m, dst_m, y1)
    y2 = _layer2_call(acc1, y1, degp, b1[None, :], W2)
    acc2 = msg_call(src_m, dst_m, y2)
    out = _final_call(acc2, y2, degp, b2[None, :], batch_p, Wo, bo[None, :])
    return out


# speedup vs baseline: 1.1195x; 1.0692x over previous
import functools

import jax
import jax.numpy as jnp
from jax import lax
from jax.experimental import pallas as pl
from jax.experimental.pallas import tpu as pltpu
from jax.experimental.pallas import tpu_sc as plsc

N = 10000
D = 128
G = 64
DO = 512
NC, NS, L = 2, 16, 16
NW = NC * NS
K = 128
NP = 10240
ROWS_PT = NP // NS
DEGW = 128
BM = 512


def _sc_mesh():
    return plsc.VectorSubcoreMesh(
        core_axis_name="c", subcore_axis_name="s", num_cores=NC, num_subcores=NS)


def _num_chunks(num_edges):
    return -(-num_edges // (NW * K))


NPR = NP // 128
DROWS_PT = 8


DBUF = 3


def _deg_body(cpt, dst_hbm, eye_hbm, degp_hbm,
              d0, d1, d2, c0, c1, c2, r0, r1, r2, zrow_v, deg_sp,
              i0, i1, i2, g0, g1, g2, x0, x1, x2):
    didx = (d0, d1, d2)
    cidx = (c0, c1, c2)
    rows = (r0, r1, r2)
    isem = (i0, i1, i2)
    gsem = (g0, g1, g2)
    ssem = (x0, x1, x2)
    c = lax.axis_index("c")
    s = lax.axis_index("s")
    wid = c * NS + s
    base = wid * (cpt * KM)

    def fill_zero(r, _):
        for j in range(DEGW // L):
            zrow_v[r, pl.ds(j * L, L)] = jnp.zeros((L,), jnp.float32)
        return 0
    lax.fori_loop(0, DROWS_PT, fill_zero, 0)

    @pl.when(s < NPR // DROWS_PT)
    def _():
        pltpu.sync_copy(zrow_v, deg_sp.at[pl.ds(s * DROWS_PT, DROWS_PT)])
    plsc.subcore_barrier()

    def i_start(t, b):
        pltpu.async_copy(dst_hbm.at[pl.ds(base + t * KM, KM)], didx[b], isem[b])

    def i_wait(t, b):
        pltpu.make_async_copy(dst_hbm.at[pl.ds(base + t * KM, KM)], didx[b],
                              isem[b]).wait()

    def split(b):
        for j in range(KM // L):
            v = didx[b][pl.ds(j * L, L)]
            cidx[b][pl.ds(j * L, L)] = lax.bitwise_and(v, 127)
            didx[b][pl.ds(j * L, L)] = lax.shift_right_logical(v, 7)

    def g_start(t, b):
        pltpu.async_copy(eye_hbm.at[cidx[b]], rows[b], gsem[b])

    def g_wait(t, b):
        pltpu.make_async_copy(eye_hbm.at[cidx[b]], rows[b], gsem[b]).wait()

    def s_start(t, b):
        pltpu.async_copy(rows[b], deg_sp.at[didx[b]], ssem[b], add=True)

    def s_wait(t, b):
        pltpu.make_async_copy(rows[b], deg_sp.at[didx[b]], ssem[b]).wait()

    i_start(0, 0)
    i_start(1, 1)
    i_wait(0, 0)
    split(0)
    g_start(0, 0)

    def grp(g, _):
        for b in range(DBUF):
            t = g * DBUF + b
            u = t + 2
            bu = (b + 2) % DBUF
            v = t + 1
            bv = (b + 1) % DBUF

            @pl.when(jnp.logical_and(u < cpt, u >= DBUF))
            def _():
                s_wait(u - DBUF, bu)
                i_start(u, bu)

            @pl.when(jnp.logical_and(u < cpt, u < DBUF))
            def _():
                i_start(u, bu)

            @pl.when(v < cpt)
            def _():
                i_wait(v, bv)
                split(bv)
                g_start(v, bv)

            g_wait(t, b)
            s_start(t, b)
        return 0
    lax.fori_loop(0, cpt // DBUF, grp, 0)
    for b in range(DBUF):
        s_wait(cpt - DBUF + b, b)
    plsc.subcore_barrier()

    @pl.when(s < NPR // DROWS_PT)
    def _():
        pltpu.sync_copy(deg_sp.at[pl.ds(s * DROWS_PT, DROWS_PT)], zrow_v)
        pltpu.sync_copy(zrow_v, degp_hbm.at[c, pl.ds(s * DROWS_PT, DROWS_PT)])


def _make_deg_call(cpt):
    return functools.partial(
        pl.kernel,
        out_type=jax.ShapeDtypeStruct((NC, NPR, DEGW), jnp.float32),
        mesh=_sc_mesh(),
        scratch_types=(
            [pltpu.VMEM((KM,), jnp.int32)] * (2 * DBUF)
            + [pltpu.VMEM((KM, DEGW), jnp.float32)] * DBUF
            + [pltpu.VMEM((DROWS_PT, DEGW), jnp.float32),
               pltpu.VMEM_SHARED((NPR, DEGW), jnp.float32)]
            + [pltpu.SemaphoreType.DMA] * (3 * DBUF)
        ),
    )(functools.partial(_deg_body, cpt))


KM = 48
MBUF = 4


def _num_chunks_msg(num_edges):
    cpt = -(-num_edges // (NW * KM))
    return -(-cpt // MBUF) * MBUF


def _msg_body(cpt, src_hbm, dst_hbm, y_hbm, accp_hbm,
              s0, s1, s2, s3, d0, d1, d2, d3, r0, r1, r2, r3, acc_sp,
              i0, i1, i2, i3, g0, g1, g2, g3, c0, c1, c2, c3):
    sidx = (s0, s1, s2, s3)
    didx = (d0, d1, d2, d3)
    rows = (r0, r1, r2, r3)
    isem = (i0, i1, i2, i3)
    gsem = (g0, g1, g2, g3)
    ssem = (c0, c1, c2, c3)
    c = lax.axis_index("c")
    s = lax.axis_index("s")
    wid = c * NS + s
    base = wid * (cpt * KM)

    def zrow(r, _):
        for j in range(D // L):
            r0[r, pl.ds(j * L, L)] = jnp.zeros((L,), jnp.float32)
        return 0
    lax.fori_loop(0, KM, zrow, 0)
    for j in range(ROWS_PT // KM):
        pltpu.sync_copy(r0, acc_sp.at[pl.ds(s * ROWS_PT + j * KM, KM)])
    rem = ROWS_PT % KM
    if rem:
        pltpu.sync_copy(r0.at[pl.ds(0, rem)],
                        acc_sp.at[pl.ds(s * ROWS_PT + (ROWS_PT // KM) * KM, rem)])
    plsc.subcore_barrier()

    def i_start(t, b):
        pltpu.async_copy(src_hbm.at[pl.ds(base + t * KM, KM)], sidx[b], isem[b])
        pltpu.async_copy(dst_hbm.at[pl.ds(base + t * KM, KM)], didx[b], isem[b])

    def i_wait(t, b):
        pltpu.make_async_copy(src_hbm.at[pl.ds(base + t * KM, KM)], sidx[b],
                              isem[b]).wait()
        pltpu.make_async_copy(dst_hbm.at[pl.ds(base + t * KM, KM)], didx[b],
                              isem[b]).wait()

    def g_start(t, b):
        pltpu.async_copy(y_hbm.at[sidx[b]], rows[b], gsem[b])

    def g_wait(t, b):
        pltpu.make_async_copy(y_hbm.at[sidx[b]], rows[b], gsem[b]).wait()

    def s_start(t, b):
        pltpu.async_copy(rows[b], acc_sp.at[didx[b]], ssem[b], add=True)

    def s_wait(t, b):
        pltpu.make_async_copy(rows[b], acc_sp.at[didx[b]], ssem[b]).wait()

    i_start(0, 0)
    i_start(1, 1)
    i_wait(0, 0)
    g_start(0, 0)

    def grp(g, _):
        for b in range(MBUF):
            t = g * MBUF + b
            u = t + 2
            bu = (b + 2) % MBUF
            v = t + 1
            bv = (b + 1) % MBUF

            @pl.when(jnp.logical_and(u < cpt, u >= MBUF))
            def _():
                s_wait(u - MBUF, bu)
                i_start(u, bu)

            @pl.when(jnp.logical_and(u < cpt, u < MBUF))
            def _():
                i_start(u, bu)

            @pl.when(v < cpt)
            def _():
                i_wait(v, bv)
                g_start(v, bv)

            g_wait(t, b)
            s_start(t, b)
        return 0
    lax.fori_loop(0, cpt // MBUF, grp, 0)
    for b in range(MBUF):
        s_wait(cpt - MBUF + b, b)
    plsc.subcore_barrier()
    for j in range(ROWS_PT // KM):
        rr = s * ROWS_PT + j * KM
        pltpu.sync_copy(acc_sp.at[pl.ds(rr, KM)], r0)
        pltpu.sync_copy(r0, accp_hbm.at[c, pl.ds(rr, KM)])
    if ROWS_PT % KM:
        rr = s * ROWS_PT + (ROWS_PT // KM) * KM
        rem = ROWS_PT % KM
        pltpu.sync_copy(acc_sp.at[pl.ds(rr, rem)], r0.at[pl.ds(0, rem)])
        pltpu.sync_copy(r0.at[pl.ds(0, rem)], accp_hbm.at[c, pl.ds(rr, rem)])


def _make_msg_call(cpt):
    return functools.partial(
        pl.kernel,
        out_type=jax.ShapeDtypeStruct((NC, NP, D), jnp.float32),
        mesh=_sc_mesh(),
        scratch_types=(
            [pltpu.VMEM((KM,), jnp.int32)] * (2 * MBUF)
            + [pltpu.VMEM((KM, D), jnp.float32)] * MBUF
            + [pltpu.VMEM_SHARED((NP, D), jnp.float32)]
            + [pltpu.SemaphoreType.DMA] * (3 * MBUF)
        ),
    )(functools.partial(_msg_body, cpt))


def _dinv_of(degp_ref):
    d = degp_ref[0] + degp_ref[1] + 1.0
    return lax.rsqrt(d)


def _scale_body(x_ref, w_ref, degp_ref, y_ref):
    dinv = _dinv_of(degp_ref)
    y_ref[...] = dinv * jnp.dot(x_ref[...], w_ref[...],
                                preferred_element_type=jnp.float32)


def _layer2_body(accp_ref, y1_ref, degp_ref, b1_ref, w2_ref, y2_ref):
    dinv = _dinv_of(degp_ref)
    h = jnp.maximum(
        dinv * (accp_ref[0] + accp_ref[1] + y1_ref[...]) + b1_ref[...], 0.0)
    y2_ref[...] = dinv * jnp.dot(h, w2_ref[...],
                                 preferred_element_type=jnp.float32)


def _final_body(nb, accp_ref, y2_ref, degp_ref, b2_ref, batch_ref, wo_ref,
                bo_ref, out_ref, sums, cnts):
    m = pl.program_id(0)

    @pl.when(m == 0)
    def _():
        sums[...] = jnp.zeros_like(sums)
        cnts[...] = jnp.zeros_like(cnts)

    dinv = _dinv_of(degp_ref)
    h = jnp.maximum(
        dinv * (accp_ref[0] + accp_ref[1] + y2_ref[...]) + b2_ref[...], 0.0)
    oh = (batch_ref[...] == lax.broadcasted_iota(jnp.int32, (G, BM), 0)
          ).astype(jnp.float32)
    sums[...] += jnp.dot(oh, h, preferred_element_type=jnp.float32)
    cnts[...] = cnts[...] + jnp.sum(oh, axis=1, keepdims=True)

    @pl.when(m == nb - 1)
    def _():
        pooled = sums[...] / jnp.maximum(cnts[...], 1.0)
        out_ref[...] = jnp.dot(pooled, wo_ref[...],
                               preferred_element_type=jnp.float32) + bo_ref[...]


_NB = NP // BM

_scale_call = pl.pallas_call(
    _scale_body,
    grid=(_NB,),
    in_specs=[
        pl.BlockSpec((BM, D), lambda i: (i, 0)),
        pl.BlockSpec((D, D), lambda i: (0, 0)),
        pl.BlockSpec((NC, BM, 1), lambda i: (0, i, 0)),
    ],
    out_specs=pl.BlockSpec((BM, D), lambda i: (i, 0)),
    out_shape=jax.ShapeDtypeStruct((NP, D), jnp.float32),
)

_layer2_call = pl.pallas_call(
    _layer2_body,
    grid=(_NB,),
    in_specs=[
        pl.BlockSpec((NC, BM, D), lambda i: (0, i, 0)),
        pl.BlockSpec((BM, D), lambda i: (i, 0)),
        pl.BlockSpec((NC, BM, 1), lambda i: (0, i, 0)),
        pl.BlockSpec((1, D), lambda i: (0, 0)),
        pl.BlockSpec((D, D), lambda i: (0, 0)),
    ],
    out_specs=pl.BlockSpec((BM, D), lambda i: (i, 0)),
    out_shape=jax.ShapeDtypeStruct((NP, D), jnp.float32),
)

_final_call = pl.pallas_call(
    functools.partial(_final_body, _NB),
    grid=(_NB,),
    in_specs=[
        pl.BlockSpec((NC, BM, D), lambda i: (0, i, 0)),
        pl.BlockSpec((BM, D), lambda i: (i, 0)),
        pl.BlockSpec((NC, BM, 1), lambda i: (0, i, 0)),
        pl.BlockSpec((1, D), lambda i: (0, 0)),
        pl.BlockSpec((1, BM), lambda i: (0, i)),
        pl.BlockSpec((D, DO), lambda i: (0, 0)),
        pl.BlockSpec((1, DO), lambda i: (0, 0)),
    ],
    out_specs=pl.BlockSpec((G, DO), lambda i: (0, 0)),
    out_shape=jax.ShapeDtypeStruct((G, DO), jnp.float32),
    scratch_shapes=[
        pltpu.VMEM((G, D), jnp.float32),
        pltpu.VMEM((G, D), jnp.float32),
    ],
)


def kernel(x, edge_index, batch, W1, b1, W2, b2, Wo, bo):
    num_edges = edge_index.shape[1]
    cptm = _num_chunks_msg(num_edges)
    cptd = -(-(-(-num_edges // (NW * KM))) // DBUF) * DBUF
    src = edge_index[0].astype(jnp.int32)
    dst = edge_index[1].astype(jnp.int32)
    padm = jnp.full((NW * cptm * KM - num_edges,), N, jnp.int32)
    padd = jnp.full((NW * cptd * KM - num_edges,), N, jnp.int32)
    dst_p = jnp.concatenate([dst, padd])
    src_m = jnp.concatenate([src, padm])
    dst_m = jnp.concatenate([dst, padm])
    n = x.shape[0]
    x_p = jnp.concatenate([x, jnp.zeros((NP - n, D), jnp.float32)])
    batch_p = jnp.concatenate(
        [batch.astype(jnp.int32), jnp.full((NP - n,), G, jnp.int32)])[None, :]

    deg_call = _make_deg_call(cptd)
    msg_call = _make_msg_call(cptm)

    eye = jnp.eye(DEGW, dtype=jnp.float32)
    degp = deg_call(dst_p, eye).reshape(NC, NP, 1)
    y1 = _scale_call(x_p, W1, degp)
    acc1 = msg_call(src_m, dst_m, y1)
    y2 = _layer2_call(acc1, y1, degp, b1[None, :], W2)
    acc2 = msg_call(src_m, dst_m, y2)
    out = _final_call(acc2, y2, degp, b2[None, :], batch_p, Wo, bo[None, :])
    return out
